# Initial kernel scaffold; baseline (speedup 1.0000x reference)
import jax
import jax.numpy as jnp
from jax.experimental import pallas as pl


def kernel(x, edge_index, edge_attr, batch, atom_emb1, atom_emb2, edge_emb1s, edge_emb2s, W1s, b1s, W2s, b2s, gammas, betas, featW, featb):
    raise NotImplementedError("write your pallas kernel here")


# speedup vs baseline: 3.1989x; 3.1989x over previous
import functools

import jax
import jax.numpy as jnp
from jax import lax
from jax.experimental import pallas as pl
from jax.experimental.pallas import tpu as pltpu
from jax.experimental.pallas import tpu_sc as plsc

N = 10000
E = 320000
G = 64
L = 5
D = 128
H = 256
FEAT = 256

NC = 2
NS = 16
DH = D // NC
NP = 10240
RPT = NP // NS
ZR = 128
T9 = 9
EH = E // 2
SEG_BIG = 10080
SEG_MED = 9840
WIN = 240
GB = 80
EPAD = 320

_mesh = plsc.VectorSubcoreMesh(core_axis_name="c", subcore_axis_name="s")
_sc_params = pltpu.CompilerParams(needs_layout_passes=False,
                                  use_tc_tiling_on_sc=False)


@functools.partial(
    pl.kernel,
    mesh=_mesh,
    out_type=jax.ShapeDtypeStruct((NC, NP, DH), jnp.float32),
    compiler_params=_sc_params,
    scratch_types=[
        pltpu.VMEM((SEG_BIG,), jnp.int32),
        pltpu.VMEM((SEG_BIG,), jnp.int32),
        pltpu.VMEM((WIN, DH), jnp.float32),
        pltpu.VMEM((256, DH), jnp.float32),
        pltpu.VMEM((16,), jnp.int32),
        pltpu.VMEM((ZR, DH), jnp.float32),
        pltpu.VMEM_SHARED((NP, DH), jnp.float32),
        pltpu.SemaphoreType.DMA,
    ],
)
def _segsum(tab_hbm, gidx_hbm, dst_hbm, out_hbm, gi_v, ds_v, rows_v, t_v,
            ix_v, zbuf_v, agg_sh, sem):
    cid = lax.axis_index("c")
    sid = lax.axis_index("s")
    i32 = jnp.int32
    f32 = jnp.float32
    zero16 = jnp.zeros((16,), f32)
    iota16 = lax.broadcasted_iota(i32, (16,), 0)

    seg_base = jnp.where(sid <= 10, sid * SEG_BIG,
                         11 * SEG_BIG + (sid - 11) * SEG_MED)
    seg_len = jnp.where(sid <= 10, SEG_BIG,
                        jnp.where(sid < 15, SEG_MED, EH - 11 * SEG_BIG
                                  - 4 * SEG_MED))
    nwin = (seg_len + WIN - 1) // WIN

    def _zt(i, c):
        t_v[i // (DH // 16), pl.ds((i % (DH // 16)) * 16, 16)] = zero16
        return c
    lax.fori_loop(0, 256 * (DH // 16), _zt, 0)

    def _zb(i, c):
        zbuf_v[i // (DH // 16), pl.ds((i % (DH // 16)) * 16, 16)] = zero16
        return c
    lax.fori_loop(0, ZR * (DH // 16), _zb, 0)
    for t in range(RPT // ZR):
        pltpu.sync_copy(zbuf_v, agg_sh.at[pl.ds(sid * RPT + t * ZR, ZR)])
    plsc.subcore_barrier()

    for half in range(2):
        off = half * EH + seg_base
        pltpu.sync_copy(gidx_hbm.at[pl.ds(off, SEG_BIG)], gi_v)
        pltpu.sync_copy(dst_hbm.at[pl.ds(off, SEG_BIG)], ds_v)

        coff = cid * (N * T9)

        def _co(i, c):
            gi_v[pl.ds(i * 16, 16)] = gi_v[pl.ds(i * 16, 16)] + coff
            return c
        lax.fori_loop(0, SEG_BIG // 16, _co, 0)

        def _win(w, base):
            wstart = w * WIN
            wlen = jnp.minimum(WIN, seg_len - wstart)
            nblk = (wlen + GB - 1) // GB

            def _g(b, c):
                pltpu.async_copy(
                    tab_hbm.at[gi_v.at[pl.ds(wstart + b * GB, GB)]],
                    rows_v.at[pl.ds(b * GB, GB)], sem)
                return c
            lax.fori_loop(0, nblk, _g, 0)

            def _gw(b, c):
                pltpu.make_async_copy(
                    tab_hbm.at[gi_v.at[pl.ds(wstart + b * GB, GB)]],
                    rows_v.at[pl.ds(b * GB, GB)], sem).wait()
                return c
            lax.fori_loop(0, nblk, _gw, 0)

            def _upd(g, c):
                dv = ds_v[pl.ds(wstart + g * 16, 16)] - base
                for j in range(16):
                    lidx = dv[j]
                    u = g * 16 + j
                    for k in range(DH // 16):
                        plsc.addupdate(t_v.at[lidx, pl.ds(k * 16, 16)],
                                       rows_v[u, pl.ds(k * 16, 16)])
                return c
            lax.fori_loop(0, wlen // 16, _upd, 0)

            last = ds_v[pl.ds(wstart + wlen - 16, 16)][15]
            complete = last - base
            nb = (complete + 15) // 16

            def _fl(b, c):
                iv = iota16 + (base + b * 16)
                iv = jnp.where(iv < last, iv, NP - 1)
                ix_v[...] = iv
                pltpu.sync_copy(t_v.at[pl.ds(b * 16, 16)],
                                agg_sh.at[ix_v], add=True)
                return c
            lax.fori_loop(0, nb, _fl, 0)

            carry = [t_v[complete, pl.ds(k * 16, 16)]
                     for k in range(DH // 16)]

            def _zf(r, c):
                for k in range(DH // 16):
                    t_v[r, pl.ds(k * 16, 16)] = zero16
                return c
            lax.fori_loop(0, nb * 16 + 1, _zf, 0)
            for k in range(DH // 16):
                t_v[0, pl.ds(k * 16, 16)] = carry[k]
            return last

        base0 = ds_v[pl.ds(0, 16)][0]
        base_end = lax.fori_loop(0, nwin, _win, base0)

        ix_v[...] = jnp.where(iota16 < 1, base_end, NP - 1)
        pltpu.sync_copy(t_v.at[pl.ds(0, 16)], agg_sh.at[ix_v], add=True)
        for k in range(DH // 16):
            t_v[0, pl.ds(k * 16, 16)] = zero16

    plsc.subcore_barrier()

    for t in range(RPT // ZR):
        base = sid * RPT + t * ZR
        pltpu.sync_copy(agg_sh.at[pl.ds(base, ZR)], zbuf_v)
        pltpu.sync_copy(zbuf_v, out_hbm.at[cid, pl.ds(base, ZR)])


def _prep_body(x_ref, e1_ref, e2_ref, h0_ref):
    x0 = x_ref[...][:, 0:1]
    x1 = x_ref[...][:, 1:2]
    zero = jnp.zeros((N, D), jnp.float32)
    s1 = zero
    s2 = zero
    for i in range(3):
        s1 = s1 + jnp.where(x0 == i, e1_ref[i], 0.0)
        s2 = s2 + jnp.where(x1 == i, e2_ref[i], 0.0)
    h0_ref[...] = s1 + s2


def _tab_body(h_ref, cb_ref, tab_ref):
    h = h_ref[...]
    for c in range(NC):
        tab_ref[c] = (h[:, None, c * DH:(c + 1) * DH]
                      + cb_ref[...][None, :, c * DH:(c + 1) * DH])


def _layer_body(part_ref, w1_ref, b1_ref, w2_ref, b2_ref, out_ref):
    agg = jnp.concatenate([part_ref[0, :N], part_ref[1, :N]], axis=1)
    t = jnp.maximum(
        jnp.dot(agg.astype(jnp.bfloat16), w1_ref[...].astype(jnp.bfloat16),
                preferred_element_type=jnp.float32)
        + b1_ref[...], 0.0)
    out_ref[...] = jnp.dot(t.astype(jnp.bfloat16),
                           w2_ref[...].astype(jnp.bfloat16),
                           preferred_element_type=jnp.float32) + b2_ref[...]


def _pool_body(h_ref, b_ref, fw_ref, fb_ref, out_ref):
    ig = lax.broadcasted_iota(jnp.int32, (N, G), 1)
    oh = (b_ref[...] == ig).astype(jnp.float32)
    sums = lax.dot_general(oh, h_ref[...], (((0,), (0,)), ((), ())),
                           preferred_element_type=jnp.float32,
                           precision=lax.Precision.HIGHEST)
    cnts = lax.dot_general(oh, jnp.ones((N, 1), jnp.float32),
                           (((0,), (0,)), ((), ())),
                           preferred_element_type=jnp.float32,
                           precision=lax.Precision.HIGHEST)
    g = sums / jnp.maximum(cnts, 1.0)
    out_ref[...] = jnp.dot(g.astype(jnp.bfloat16),
                           fw_ref[...].astype(jnp.bfloat16),
                           preferred_element_type=jnp.float32) + fb_ref[...]


def kernel(x, edge_index, edge_attr, batch, atom_emb1, atom_emb2,
           edge_emb1s, edge_emb2s, W1s, b1s, W2s, b2s, gammas, betas,
           featW, featb):
    f32 = jnp.float32
    i32 = jnp.int32
    dst_e = edge_index[1]
    order = jnp.argsort(dst_e, stable=True)
    gbase = (edge_index[0] * T9 + edge_attr[:, 0] * 3
             + edge_attr[:, 1]).astype(i32)
    gidx_s = jnp.pad(gbase[order], (0, EPAD))
    dst_s = jnp.pad(dst_e[order].astype(i32), (0, EPAD))
    combos = (edge_emb1s[:, :3, None, :]
              + edge_emb2s[:, None, :3, :]).reshape(L, T9, D)

    h = pl.pallas_call(
        _prep_body,
        out_shape=jax.ShapeDtypeStruct((N, D), f32),
    )(x, atom_emb1[:3], atom_emb2[:3])

    for l in range(L):
        tab = pl.pallas_call(
            _tab_body,
            grid=(10,),
            in_specs=[pl.BlockSpec((N // 10, D), lambda i: (i, 0)),
                      pl.BlockSpec((T9, D), lambda i: (0, 0))],
            out_specs=pl.BlockSpec((NC, N // 10, T9, DH),
                                   lambda i: (0, i, 0, 0)),
            out_shape=jax.ShapeDtypeStruct((NC, N, T9, DH), f32),
        )(h, combos[l])
        part = _segsum(tab.reshape(NC * N * T9, DH), gidx_s, dst_s)
        h2 = pl.pallas_call(
            _layer_body,
            out_shape=jax.ShapeDtypeStruct((N, D), f32),
        )(part, W1s[l], b1s[l].reshape(1, H), W2s[l], b2s[l].reshape(1, D))
        agg = jnp.concatenate([part[0, :N], part[1, :N]], axis=1)
        h2s = jnp.maximum(agg @ W1s[l] + b1s[l], 0.0) @ W2s[l] + b2s[l]
        mu = jnp.mean(h2s, axis=0)
        var = jnp.var(h2s, axis=0)
        h2 = gammas[l] * (h2 - mu) / jnp.sqrt(var + 1e-5) + betas[l]
        if l != L - 1:
            h2 = jax.nn.relu(h2)
        h = h2

    return pl.pallas_call(
        _pool_body,
        out_shape=jax.ShapeDtypeStruct((G, FEAT), f32),
    )(h, batch.reshape(N, 1).astype(i32), featW, featb.reshape(1, FEAT))
